# baseline (speedup 1.0000x reference)
import functools

import jax
import jax.numpy as jnp
from jax import lax
from jax.experimental import pallas as pl
from jax.experimental.pallas import tpu as pltpu
from jax.experimental.pallas import tpu_sc as plsc

_B, _C, _N, _K = 16, 768, 2048, 128
_R = _B * _C
_NW = 32
_ROWS_PER_W = _R // _NW
_VPC = _K // 16


def _sel(c, a, b):
  return jnp.where(c, a, b)


def _bitonic_clean(vs, sortfn, descending):
  vs = list(vs)
  m = len(vs)
  s = m // 2
  while s >= 1:
    nxt = list(vs)
    for blk in range(0, m, 2 * s):
      for i in range(blk, blk + s):
        a, b = vs[i], vs[i + s]
        if descending:
          nxt[i] = jnp.maximum(a, b)
          nxt[i + s] = jnp.minimum(a, b)
        else:
          nxt[i] = jnp.minimum(a, b)
          nxt[i + s] = jnp.maximum(a, b)
    vs = nxt
    s //= 2
  return [sortfn(v, descending) for v in vs]


def _merge_keep_all(run_a, run_b, sortfn, descending):
  m = len(run_a)
  if descending:
    hi = [jnp.maximum(run_a[i], run_b[i]) for i in range(m)]
    lo = [jnp.minimum(run_a[i], run_b[i]) for i in range(m)]
  else:
    hi = [jnp.minimum(run_a[i], run_b[i]) for i in range(m)]
    lo = [jnp.maximum(run_a[i], run_b[i]) for i in range(m)]
  return (_bitonic_clean(hi, sortfn, descending)
          + _bitonic_clean(lo, sortfn, descending))


def _merge_keep_top(run_a, run_b, sortfn):
  m = len(run_a)
  hi = [jnp.maximum(run_a[i], run_b[i]) for i in range(m)]
  return _bitonic_clean(hi, sortfn, True)


def _build_sorted_run(vs, sortfn, descending):
  if len(vs) == 1:
    return [sortfn(vs[0], descending)]
  h = len(vs) // 2
  run_a = _build_sorted_run(vs[:h], sortfn, descending)
  run_b = _build_sorted_run(vs[h:], sortfn, not descending)
  return _merge_keep_all(run_a, run_b, sortfn, descending)


def _kv_clean(ks, vs, sortkv, descending):
  ks, vs = list(ks), list(vs)
  m = len(ks)
  s = m // 2
  while s >= 1:
    nk, nv = list(ks), list(vs)
    for blk in range(0, m, 2 * s):
      for i in range(blk, blk + s):
        c = ks[i] >= ks[i + s] if descending else ks[i] <= ks[i + s]
        nk[i] = _sel(c, ks[i], ks[i + s])
        nv[i] = _sel(c, vs[i], vs[i + s])
        nk[i + s] = _sel(c, ks[i + s], ks[i])
        nv[i + s] = _sel(c, vs[i + s], vs[i])
    ks, vs = nk, nv
    s //= 2
  pairs = [sortkv(ks[i], vs[i], descending) for i in range(m)]
  return [p[0] for p in pairs], [p[1] for p in pairs]


def _kv_merge_keep_all(ka, va, kb, vb, sortkv, descending):
  m = len(ka)
  hk, hv, lk, lv = [], [], [], []
  for i in range(m):
    c = ka[i] >= kb[i] if descending else ka[i] <= kb[i]
    hk.append(_sel(c, ka[i], kb[i]))
    hv.append(_sel(c, va[i], vb[i]))
    lk.append(_sel(c, kb[i], ka[i]))
    lv.append(_sel(c, vb[i], va[i]))
  hk, hv = _kv_clean(hk, hv, sortkv, descending)
  lk, lv = _kv_clean(lk, lv, sortkv, descending)
  return hk + lk, hv + lv


def _kv_keep_top(ka, va, kb, vb, sortkv):
  m = len(ka)
  hk, hv = [], []
  for i in range(m):
    c = ka[i] >= kb[i]
    hk.append(_sel(c, ka[i], kb[i]))
    hv.append(_sel(c, va[i], vb[i]))
  return _kv_clean(hk, hv, sortkv, True)


def _kv_build(ks, vs, sortkv, descending):
  if len(ks) == 1:
    k2, v2 = sortkv(ks[0], vs[0], descending)
    return [k2], [v2]
  h = len(ks) // 2
  ka, va = _kv_build(ks[:h], vs[:h], sortkv, descending)
  kb, vb = _kv_build(ks[h:], vs[h:], sortkv, not descending)
  return _kv_merge_keep_all(ka, va, kb, vb, sortkv, descending)


_BASE1, _BASE2, _BASE3 = 0, 1024, 1536


def _row_core(load_row, load_scr, store_scr, gather_row, gather_scr,
              sortk, sortkv, iota16):
  for i in range(64):
    store_scr(i, jnp.maximum(load_row(i), load_row(i + 64)))
  for i in range(32):
    store_scr(64 + i, jnp.maximum(load_scr(i), load_scr(i + 32)))
  for i in range(16):
    store_scr(96 + i, jnp.maximum(load_scr(64 + i), load_scr(80 + i)))
  a4 = [jnp.maximum(load_scr(96 + i), load_scr(104 + i)) for i in range(8)]

  pos0 = [iota16 + 16 * i for i in range(8)]
  ks, vs = _kv_build(a4, pos0, sortkv, True)

  levels = ((gather_scr, _BASE3, 128, True),
            (gather_scr, _BASE2, 256, True),
            (gather_scr, _BASE1, 512, True),
            (gather_row, 0, 1024, False))
  for gather, base, half, want_kv in levels:
    pk, pv, nv = [], [], []
    for i in range(8):
      a = gather(base + vs[i])
      b = gather(base + vs[i] + half)
      c = a >= b
      pk.append(jnp.minimum(a, b))
      if want_kv:
        pv.append(_sel(c, vs[i] + half, vs[i]))
        nv.append(_sel(c, vs[i], vs[i] + half))
    if want_kv:
      bk, bv = _kv_build(pk, pv, sortkv, False)
      ks, vs = _kv_keep_top(ks, nv, bk, bv, sortkv)
    else:
      brun = _build_sorted_run(pk, sortk, False)
      ks = _merge_keep_top(ks, brun, sortk)
  return ks


def _sc_sortk(v, descending):
  return plsc.sort_key_val(v, v, descending=descending)[0]


def _sc_sortkv(k, v, descending):
  return plsc.sort_key_val(k, v, descending=descending)


@functools.lru_cache(maxsize=1)
def _make_sc_topk():
  mesh = plsc.VectorSubcoreMesh(
      core_axis_name="c", subcore_axis_name="s", num_cores=2, num_subcores=16)

  @functools.partial(
      pl.kernel,
      out_type=jax.ShapeDtypeStruct((_R, _K), jnp.float32),
      mesh=mesh,
      scratch_types=[
          pltpu.VMEM((_N,), jnp.float32),
          pltpu.VMEM((_N,), jnp.float32),
          pltpu.VMEM((1792,), jnp.float32),
          pltpu.VMEM((_K,), jnp.float32),
          pltpu.VMEM((_K,), jnp.float32),
          pltpu.SemaphoreType.DMA,
          pltpu.SemaphoreType.DMA,
          pltpu.SemaphoreType.DMA,
          pltpu.SemaphoreType.DMA,
      ],
      compiler_params=pltpu.CompilerParams(needs_layout_passes=False),
  )
  def topk_rows(x_hbm, out_hbm, in_a, in_b, scr, o0, o1,
                sem_a, sem_b, sem_o0, sem_o1):
    wid = lax.axis_index("s") * 2 + lax.axis_index("c")
    base = wid * _ROWS_PER_W
    iota16 = lax.iota(jnp.int32, 16)

    def compute(buf, out_buf):
      run = _row_core(
          lambda i: buf[pl.ds(16 * i, 16)],
          lambda s: scr[pl.ds(16 * s, 16)],
          lambda s, v: scr.__setitem__(pl.ds(16 * s, 16), v),
          lambda idx: plsc.load_gather(buf, [idx]),
          lambda idx: plsc.load_gather(scr, [idx]),
          _sc_sortk, _sc_sortkv, iota16)
      for i in range(_VPC):
        out_buf[pl.ds(16 * i, 16)] = run[i]

    pltpu.sync_copy(x_hbm.at[base], in_a)

    def body(q, carry):
      r0 = base + 2 * q
      dma_b = pltpu.async_copy(x_hbm.at[r0 + 1], in_b, sem_b)

      @pl.when(q > 0)
      def _wait_o0():
        pltpu.make_async_copy(o0, out_hbm.at[r0], sem_o0).wait()

      compute(in_a, o0)
      pltpu.async_copy(o0, out_hbm.at[r0], sem_o0)
      dma_b.wait()

      nxt = jnp.minimum(r0 + 2, _R - 1)
      dma_a = pltpu.async_copy(x_hbm.at[nxt], in_a, sem_a)

      @pl.when(q > 0)
      def _wait_o1():
        pltpu.make_async_copy(o1, out_hbm.at[r0], sem_o1).wait()

      compute(in_b, o1)
      pltpu.async_copy(o1, out_hbm.at[r0 + 1], sem_o1)
      dma_a.wait()
      return carry

    lax.fori_loop(0, _ROWS_PER_W // 2, body, 0)
    pltpu.make_async_copy(o0, out_hbm.at[base], sem_o0).wait()
    pltpu.make_async_copy(o1, out_hbm.at[base], sem_o1).wait()

  return topk_rows


def kernel(x):
  rows = x.reshape(_R, _N)
  out = _make_sc_topk()(rows)
  return out.reshape(_B, _C, _K)
